# baseline (speedup 1.0000x reference)
import functools
import math

import jax
import jax.numpy as jnp
import numpy as np
from jax.experimental import pallas as pl
from jax.experimental.pallas import tpu as pltpu

_NUM_ANGLE = 100
_NUM_RHO = 100
_A_BLK = 5


@functools.lru_cache(maxsize=None)
def _rho_table(H, W):
    irho = int(math.sqrt(H * H + W * W) + 1) / float(_NUM_RHO)
    itheta = math.pi / _NUM_ANGLE
    angles = np.arange(_NUM_ANGLE, dtype=np.float64) * itheta
    cosv = (np.cos(angles) / irho).astype(np.float32)
    sinv = (np.sin(angles) / irho).astype(np.float32)
    ys, xs = np.meshgrid(np.arange(H), np.arange(W), indexing="ij")
    xx = (xs - W // 2).reshape(-1).astype(np.float32)
    yy = (ys - H // 2).reshape(-1).astype(np.float32)
    r = np.round(xx[None, :] * cosv[:, None] + yy[None, :] * sinv[:, None])
    r = r.astype(np.int32) + _NUM_RHO // 2
    r = np.clip(r, 0, _NUM_RHO - 1)
    nblk = _NUM_ANGLE // _A_BLK
    return r.reshape(nblk, _A_BLK, H * W)


def _build_onehot(ridx_blk, oh_ref):
    hw = ridx_blk.shape[-1]
    rho = jax.lax.broadcasted_iota(jnp.int32, (_NUM_RHO, hw), 0)
    for j in range(_A_BLK):
        row = ridx_blk[j, :].reshape(1, hw)
        oh_ref[j * _NUM_RHO : (j + 1) * _NUM_RHO, :] = (row == rho).astype(
            jnp.bfloat16
        )


def _dht_body(ridx_cur_ref, ridx_next_ref, x_ref, out_ref, xbf_ref, oh0_ref, oh1_ref):
    i = pl.program_id(0)
    nblk = pl.num_programs(0)
    bufs = (oh0_ref, oh1_ref)

    @pl.when(i == 0)
    def _():
        xbf_ref[...] = x_ref[...].astype(jnp.bfloat16)
        _build_onehot(ridx_cur_ref[0], bufs[0])

    for par in (0, 1):

        @pl.when(jax.lax.rem(i, 2) == par)
        def _():
            out_ref[0] = jax.lax.dot_general(
                xbf_ref[...],
                bufs[par][...],
                dimension_numbers=(((1,), (1,)), ((), ())),
                preferred_element_type=jnp.float32,
            )

    @pl.when(i + 1 < nblk)
    def _():
        for par in (0, 1):

            @pl.when(jax.lax.rem(i + 1, 2) == par)
            def _():
                _build_onehot(ridx_next_ref[0], bufs[par])


def kernel(x):
    B, C, H, W = x.shape
    BC = B * C
    HW = H * W
    nblk = _NUM_ANGLE // _A_BLK
    ridx = jnp.asarray(_rho_table(H, W))
    xf = x.reshape(BC, HW)

    out = pl.pallas_call(
        _dht_body,
        grid=(nblk,),
        in_specs=[
            pl.BlockSpec((1, _A_BLK, HW), lambda i: (i, 0, 0)),
            pl.BlockSpec(
                (1, _A_BLK, HW), lambda i: (jnp.minimum(i + 1, nblk - 1), 0, 0)
            ),
            pl.BlockSpec((BC, HW), lambda i: (0, 0)),
        ],
        out_specs=pl.BlockSpec((1, BC, _A_BLK * _NUM_RHO), lambda i: (i, 0, 0)),
        out_shape=jax.ShapeDtypeStruct((nblk, BC, _A_BLK * _NUM_RHO), jnp.float32),
        scratch_shapes=[
            pltpu.VMEM((BC, HW), jnp.bfloat16),
            pltpu.VMEM((_A_BLK * _NUM_RHO, HW), jnp.bfloat16),
            pltpu.VMEM((_A_BLK * _NUM_RHO, HW), jnp.bfloat16),
        ],
    )(ridx, ridx, xf)

    out = out.reshape(nblk, BC, _A_BLK, _NUM_RHO)
    acc = jnp.transpose(out, (1, 0, 2, 3)).reshape(BC, _NUM_ANGLE, _NUM_RHO)
    return acc.reshape(B, C, _NUM_ANGLE, _NUM_RHO)


# speedup vs baseline: 1.3576x; 1.3576x over previous
import functools
import math

import jax
import jax.numpy as jnp
import numpy as np
from jax.experimental import pallas as pl
from jax.experimental.pallas import tpu as pltpu

_NUM_ANGLE = 100
_NUM_RHO = 100
_A_BLK = 5
_HALF = _NUM_ANGLE // 2


@functools.lru_cache(maxsize=None)
def _rho_table(H, W):
    irho = int(math.sqrt(H * H + W * W) + 1) / float(_NUM_RHO)
    itheta = math.pi / _NUM_ANGLE
    angles = np.arange(_NUM_ANGLE, dtype=np.float64) * itheta
    cosv = (np.cos(angles) / irho).astype(np.float32)
    sinv = (np.sin(angles) / irho).astype(np.float32)
    ys, xs = np.meshgrid(np.arange(H), np.arange(W), indexing="ij")
    xx = (xs - W // 2).reshape(-1).astype(np.float32)
    yy = (ys - H // 2).reshape(-1).astype(np.float32)
    r = np.round(xx[None, :] * cosv[:, None] + yy[None, :] * sinv[:, None])
    r = r.astype(np.int32) + _NUM_RHO // 2
    return np.clip(r, 0, _NUM_RHO - 1)


@functools.lru_cache(maxsize=None)
def _tables(H, W):
    r = _rho_table(H, W)
    HW = H * W
    nblk = _HALF // _A_BLK
    lo = r[:_HALF]
    onehot = lo[:, None, :] == np.arange(_NUM_RHO, dtype=np.int32)[None, :, None]
    onehot = onehot.reshape(nblk, _A_BLK * _NUM_RHO, HW).astype(jnp.bfloat16)
    hi = np.ascontiguousarray(r[_HALF:]).reshape(nblk, _A_BLK, HW)
    return onehot, hi


def _dht_body(oh_ref, ridx_ref, x_ref, out_lo_ref, out_hi_ref, xbf_ref):
    @pl.when(pl.program_id(0) == 0)
    def _():
        xbf_ref[...] = x_ref[...].astype(jnp.bfloat16)

    hw = x_ref.shape[1]
    xbf = xbf_ref[...]

    out_lo_ref[0] = jax.lax.dot_general(
        xbf,
        oh_ref[0],
        dimension_numbers=(((1,), (1,)), ((), ())),
        preferred_element_type=jnp.float32,
    )

    rho = jax.lax.broadcasted_iota(jnp.int32, (_NUM_RHO, hw), 0)
    parts = []
    for j in range(_A_BLK):
        row = ridx_ref[0, j, :].reshape(1, hw)
        parts.append((row == rho).astype(jnp.bfloat16))
    oh_built = jnp.concatenate(parts, axis=0)
    out_hi_ref[0] = jax.lax.dot_general(
        xbf,
        oh_built,
        dimension_numbers=(((1,), (1,)), ((), ())),
        preferred_element_type=jnp.float32,
    )


def kernel(x):
    B, C, H, W = x.shape
    BC = B * C
    HW = H * W
    nblk = _HALF // _A_BLK
    onehot_np, ridx_np = _tables(H, W)
    onehot = jnp.asarray(onehot_np)
    ridx = jnp.asarray(ridx_np)
    xf = x.reshape(BC, HW)

    out_lo, out_hi = pl.pallas_call(
        _dht_body,
        grid=(nblk,),
        in_specs=[
            pl.BlockSpec((1, _A_BLK * _NUM_RHO, HW), lambda i: (i, 0, 0)),
            pl.BlockSpec((1, _A_BLK, HW), lambda i: (i, 0, 0)),
            pl.BlockSpec((BC, HW), lambda i: (0, 0)),
        ],
        out_specs=[
            pl.BlockSpec((1, BC, _A_BLK * _NUM_RHO), lambda i: (i, 0, 0)),
            pl.BlockSpec((1, BC, _A_BLK * _NUM_RHO), lambda i: (i, 0, 0)),
        ],
        out_shape=[
            jax.ShapeDtypeStruct((nblk, BC, _A_BLK * _NUM_RHO), jnp.float32),
            jax.ShapeDtypeStruct((nblk, BC, _A_BLK * _NUM_RHO), jnp.float32),
        ],
        scratch_shapes=[pltpu.VMEM((BC, HW), jnp.bfloat16)],
    )(onehot, ridx, xf)

    def _asm(o):
        o = o.reshape(nblk, BC, _A_BLK, _NUM_RHO)
        return jnp.transpose(o, (1, 0, 2, 3)).reshape(BC, _HALF, _NUM_RHO)

    acc = jnp.concatenate([_asm(out_lo), _asm(out_hi)], axis=1)
    return acc.reshape(B, C, _NUM_ANGLE, _NUM_RHO)
